# baseline (device time: 15754 ns/iter reference)
import jax
import jax.numpy as jnp
from jax import lax
from jax.experimental import pallas as pl
from jax.experimental.pallas import tpu as pltpu

N_DEV = 4
B, SQ, SKV, HQ_LOCAL, DH = 2, 256, 256, 4, 64
DMODEL = 512
WINDOW = 128
SCALE = 0.125
CH = 4
QR = SQ // N_DEV
CW = DMODEL // CH
NP = N_DEV - 1


def kernel(x, Wq, K_ext, V_ext, Wo):
    my = lax.axis_index("i")
    K_sh = lax.dynamic_slice_in_dim(K_ext, my * HQ_LOCAL, HQ_LOCAL, axis=2)
    V_sh = lax.dynamic_slice_in_dim(V_ext, my * HQ_LOCAL, HQ_LOCAL, axis=2)
    K_sh = K_sh.reshape(B, SKV, HQ_LOCAL * DH)
    V_sh = V_sh.reshape(B, SKV, HQ_LOCAL * DH)

    def body(x_ref, wq_ref, k_ref, v_ref, wo_ref, out_ref,
             ctx_ref, part_ref, fin_ref, scat_ref,
             send_a, recv_a, send_b, recv_b):
        me = lax.axis_index("i")

        barrier_sem = pltpu.get_barrier_semaphore()
        for d in range(1, N_DEV):
            pl.semaphore_signal(
                barrier_sem, inc=1,
                device_id=((me + d) % N_DEV,),
                device_id_type=pl.DeviceIdType.MESH,
            )

        qi = lax.broadcasted_iota(jnp.int32, (SQ, SKV), 0)
        ki = lax.broadcasted_iota(jnp.int32, (SQ, SKV), 1)
        mask = jnp.abs(qi - ki) <= WINDOW

        wq = wq_ref[:, :].astype(jnp.bfloat16)
        wo = wo_ref[:, :].astype(jnp.bfloat16)
        for b in range(B):
            q_b = jnp.dot(x_ref[b].astype(jnp.bfloat16), wq,
                          preferred_element_type=jnp.float32)
            q_b16 = q_b.astype(jnp.bfloat16)
            k_all = k_ref[b].astype(jnp.bfloat16)
            v_all = v_ref[b].astype(jnp.bfloat16)
            for h in range(HQ_LOCAL):
                q_bh = q_b16[:, h * DH:(h + 1) * DH]
                k_bh = k_all[:, h * DH:(h + 1) * DH]
                s = lax.dot_general(
                    q_bh, k_bh, (((1,), (1,)), ((), ())),
                    preferred_element_type=jnp.float32,
                ) * SCALE
                w = jnp.where(mask, jnp.exp(s), 0.0)
                w = w / jnp.sum(w, axis=1, keepdims=True)
                ctx_ref[b, :, h * DH:(h + 1) * DH] = jnp.dot(
                    w.astype(jnp.bfloat16), v_all[:, h * DH:(h + 1) * DH],
                    preferred_element_type=jnp.float32,
                ).astype(jnp.bfloat16)
        for b in range(B):
            p_b = jnp.dot(ctx_ref[b], wo, preferred_element_type=jnp.float32)
            out_ref[b] = p_b
            part_ref[b] = p_b.astype(jnp.bfloat16)

        my_rows = pl.ds(me * QR, QR)

        pl.semaphore_wait(barrier_sem, N_DEV - 1)

        sends = []
        for c in range(CH):
            cols = pl.ds(c * CW, CW)
            for d in range(1, N_DEV):
                t = (me + d) % N_DEV
                rdma = pltpu.make_async_remote_copy(
                    src_ref=part_ref.at[:, pl.ds(t * QR, QR), cols],
                    dst_ref=scat_ref.at[N_DEV - 1 - d, :, :, cols],
                    send_sem=send_a.at[c * NP + d - 1],
                    recv_sem=recv_a.at[c * NP + N_DEV - 1 - d],
                    device_id=(t,),
                    device_id_type=pl.DeviceIdType.MESH,
                )
                rdma.start()
                sends.append(rdma)

        for c in range(CH):
            cols = pl.ds(c * CW, CW)
            for r in range(NP):
                pltpu.make_async_remote_copy(
                    src_ref=scat_ref.at[r, :, :, cols],
                    dst_ref=scat_ref.at[r, :, :, cols],
                    send_sem=send_a.at[c * NP + r],
                    recv_sem=recv_a.at[c * NP + r],
                    device_id=(me,), device_id_type=pl.DeviceIdType.MESH,
                ).wait_recv()
            red = (
                out_ref[:, my_rows, cols]
                + scat_ref[0, :, :, cols].astype(jnp.float32)
                + scat_ref[1, :, :, cols].astype(jnp.float32)
                + scat_ref[2, :, :, cols].astype(jnp.float32)
            )
            out_ref[:, my_rows, cols] = red
            fin_ref[:, my_rows, cols] = red.astype(jnp.bfloat16)
            for d in range(1, N_DEV):
                t = (me + d) % N_DEV
                rdma = pltpu.make_async_remote_copy(
                    src_ref=fin_ref.at[:, my_rows, cols],
                    dst_ref=fin_ref.at[:, my_rows, cols],
                    send_sem=send_b.at[c * NP + d - 1],
                    recv_sem=recv_b.at[c * NP + N_DEV - 1 - d],
                    device_id=(t,),
                    device_id_type=pl.DeviceIdType.MESH,
                )
                rdma.start()
                sends.append(rdma)

        for c in range(CH):
            cols = pl.ds(c * CW, CW)
            for r in range(NP):
                s = (me + 1 + r) % N_DEV
                pltpu.make_async_remote_copy(
                    src_ref=fin_ref.at[:, pl.ds(s * QR, QR), cols],
                    dst_ref=fin_ref.at[:, pl.ds(s * QR, QR), cols],
                    send_sem=send_b.at[c * NP + r],
                    recv_sem=recv_b.at[c * NP + r],
                    device_id=(me,), device_id_type=pl.DeviceIdType.MESH,
                ).wait_recv()
        for r in range(NP):
            s_rows = pl.ds(((me + 1 + r) % N_DEV) * QR, QR)
            out_ref[:, s_rows, :] = fin_ref[:, s_rows, :].astype(jnp.float32)
        for rdma in sends:
            rdma.wait_send()

    return pl.pallas_call(
        body,
        out_shape=jax.ShapeDtypeStruct((B, SQ, DMODEL), jnp.float32),
        in_specs=[pl.BlockSpec(memory_space=pltpu.VMEM)] * 5,
        out_specs=pl.BlockSpec(memory_space=pltpu.VMEM),
        scratch_shapes=[
            pltpu.VMEM((B, SQ, HQ_LOCAL * DH), jnp.bfloat16),
            pltpu.VMEM((B, SQ, DMODEL), jnp.bfloat16),
            pltpu.VMEM((B, SQ, DMODEL), jnp.bfloat16),
            pltpu.VMEM((NP, B, QR, DMODEL), jnp.bfloat16),
            pltpu.SemaphoreType.DMA((CH * NP,)),
            pltpu.SemaphoreType.DMA((CH * NP,)),
            pltpu.SemaphoreType.DMA((CH * NP,)),
            pltpu.SemaphoreType.DMA((CH * NP,)),
        ],
        compiler_params=pltpu.CompilerParams(collective_id=0),
    )(x, Wq, K_sh, V_sh, Wo)


# device time: 15231 ns/iter; 1.0343x vs baseline; 1.0343x over previous
import jax
import jax.numpy as jnp
from jax import lax
from jax.experimental import pallas as pl
from jax.experimental.pallas import tpu as pltpu

N_DEV = 4
B, SQ, SKV, HQ_LOCAL, DH = 2, 256, 256, 4, 64
DMODEL = 512
WINDOW = 128
SCALE = 0.125
CH = 4
QR = SQ // N_DEV
CW = DMODEL // CH
NP = N_DEV - 1


def kernel(x, Wq, K_ext, V_ext, Wo):
    my = lax.axis_index("i")
    K_sh = lax.dynamic_slice_in_dim(K_ext, my * HQ_LOCAL, HQ_LOCAL, axis=2)
    V_sh = lax.dynamic_slice_in_dim(V_ext, my * HQ_LOCAL, HQ_LOCAL, axis=2)
    K_sh = jnp.transpose(K_sh, (0, 2, 1, 3)).astype(jnp.bfloat16)
    V_sh = jnp.transpose(V_sh, (0, 2, 1, 3)).astype(jnp.bfloat16)

    def body(x_ref, wq_ref, k_ref, v_ref, wo_ref, out_ref,
             ctx_ref, part_ref, fin_ref, scat_ref,
             send_a, recv_a, send_b, recv_b):
        me = lax.axis_index("i")

        barrier_sem = pltpu.get_barrier_semaphore()
        for d in range(1, N_DEV):
            pl.semaphore_signal(
                barrier_sem, inc=1,
                device_id=((me + d) % N_DEV,),
                device_id_type=pl.DeviceIdType.MESH,
            )

        qi = lax.broadcasted_iota(jnp.int32, (SQ, SKV), 0)
        ki = lax.broadcasted_iota(jnp.int32, (SQ, SKV), 1)
        mask = jnp.abs(qi - ki) <= WINDOW

        wq = wq_ref[:, :].astype(jnp.bfloat16)
        wo = wo_ref[:, :].astype(jnp.bfloat16)
        my_rows = pl.ds(me * QR, QR)
        sends = []

        def sem_idx(b, c, j):
            return (b * CH + c) * NP + j

        for b in range(B):
            q_b = jnp.dot(x_ref[b].astype(jnp.bfloat16), wq,
                          preferred_element_type=jnp.float32)
            q_b16 = q_b.astype(jnp.bfloat16)
            for h in range(HQ_LOCAL):
                q_bh = q_b16[:, h * DH:(h + 1) * DH]
                k_bh = k_ref[b, h]
                s = lax.dot_general(
                    q_bh, k_bh, (((1,), (1,)), ((), ())),
                    preferred_element_type=jnp.float32,
                ) * SCALE
                w = jnp.where(mask, jnp.exp(s), 0.0)
                w = w / jnp.sum(w, axis=1, keepdims=True)
                ctx_ref[b, :, h * DH:(h + 1) * DH] = jnp.dot(
                    w.astype(jnp.bfloat16), v_ref[b, h],
                    preferred_element_type=jnp.float32,
                ).astype(jnp.bfloat16)
            p_b = jnp.dot(ctx_ref[b], wo, preferred_element_type=jnp.float32)
            out_ref[b] = p_b
            part_ref[b] = p_b.astype(jnp.bfloat16)

            if b == 0:
                pl.semaphore_wait(barrier_sem, N_DEV - 1)
            for c in range(CH):
                cols = pl.ds(c * CW, CW)
                for d in range(1, N_DEV):
                    t = (me + d) % N_DEV
                    rdma = pltpu.make_async_remote_copy(
                        src_ref=part_ref.at[b, pl.ds(t * QR, QR), cols],
                        dst_ref=scat_ref.at[N_DEV - 1 - d, b, :, cols],
                        send_sem=send_a.at[sem_idx(b, c, d - 1)],
                        recv_sem=recv_a.at[sem_idx(b, c, N_DEV - 1 - d)],
                        device_id=(t,),
                        device_id_type=pl.DeviceIdType.MESH,
                    )
                    rdma.start()
                    sends.append(rdma)

        for b in range(B):
            for c in range(CH):
                cols = pl.ds(c * CW, CW)
                for r in range(NP):
                    pltpu.make_async_remote_copy(
                        src_ref=scat_ref.at[r, b, :, cols],
                        dst_ref=scat_ref.at[r, b, :, cols],
                        send_sem=send_a.at[sem_idx(b, c, r)],
                        recv_sem=recv_a.at[sem_idx(b, c, r)],
                        device_id=(me,), device_id_type=pl.DeviceIdType.MESH,
                    ).wait_recv()
                red = (
                    out_ref[b, my_rows, cols]
                    + scat_ref[0, b, :, cols].astype(jnp.float32)
                    + scat_ref[1, b, :, cols].astype(jnp.float32)
                    + scat_ref[2, b, :, cols].astype(jnp.float32)
                )
                out_ref[b, my_rows, cols] = red
                fin_ref[b, my_rows, cols] = red.astype(jnp.bfloat16)
                for d in range(1, N_DEV):
                    t = (me + d) % N_DEV
                    rdma = pltpu.make_async_remote_copy(
                        src_ref=fin_ref.at[b, my_rows, cols],
                        dst_ref=fin_ref.at[b, my_rows, cols],
                        send_sem=send_b.at[sem_idx(b, c, d - 1)],
                        recv_sem=recv_b.at[sem_idx(b, c, N_DEV - 1 - d)],
                        device_id=(t,),
                        device_id_type=pl.DeviceIdType.MESH,
                    )
                    rdma.start()
                    sends.append(rdma)

        for b in range(B):
            for c in range(CH):
                cols = pl.ds(c * CW, CW)
                for r in range(NP):
                    s = (me + 1 + r) % N_DEV
                    pltpu.make_async_remote_copy(
                        src_ref=fin_ref.at[b, pl.ds(s * QR, QR), cols],
                        dst_ref=fin_ref.at[b, pl.ds(s * QR, QR), cols],
                        send_sem=send_b.at[sem_idx(b, c, r)],
                        recv_sem=recv_b.at[sem_idx(b, c, r)],
                        device_id=(me,), device_id_type=pl.DeviceIdType.MESH,
                    ).wait_recv()
        for r in range(NP):
            s_rows = pl.ds(((me + 1 + r) % N_DEV) * QR, QR)
            out_ref[:, s_rows, :] = fin_ref[:, s_rows, :].astype(jnp.float32)
        for rdma in sends:
            rdma.wait_send()

    return pl.pallas_call(
        body,
        out_shape=jax.ShapeDtypeStruct((B, SQ, DMODEL), jnp.float32),
        in_specs=[pl.BlockSpec(memory_space=pltpu.VMEM)] * 5,
        out_specs=pl.BlockSpec(memory_space=pltpu.VMEM),
        scratch_shapes=[
            pltpu.VMEM((B, SQ, HQ_LOCAL * DH), jnp.bfloat16),
            pltpu.VMEM((B, SQ, DMODEL), jnp.bfloat16),
            pltpu.VMEM((B, SQ, DMODEL), jnp.bfloat16),
            pltpu.VMEM((NP, B, QR, DMODEL), jnp.bfloat16),
            pltpu.SemaphoreType.DMA((B * CH * NP,)),
            pltpu.SemaphoreType.DMA((B * CH * NP,)),
            pltpu.SemaphoreType.DMA((B * CH * NP,)),
            pltpu.SemaphoreType.DMA((B * CH * NP,)),
        ],
        compiler_params=pltpu.CompilerParams(collective_id=0),
    )(x, Wq, K_sh, V_sh, Wo)


# device time: 15205 ns/iter; 1.0361x vs baseline; 1.0017x over previous
import jax
import jax.numpy as jnp
from jax import lax
from jax.experimental import pallas as pl
from jax.experimental.pallas import tpu as pltpu

N_DEV = 4
B, SQ, SKV, HQ_LOCAL, DH = 2, 256, 256, 4, 64
DMODEL = 512
WINDOW = 128
SCALE = 0.125
CH = 4
QR = SQ // N_DEV
CW = DMODEL // CH
NP = N_DEV - 1


def kernel(x, Wq, K_ext, V_ext, Wo):
    my = lax.axis_index("i")
    K_sh = lax.dynamic_slice_in_dim(K_ext, my * HQ_LOCAL, HQ_LOCAL, axis=2)
    V_sh = lax.dynamic_slice_in_dim(V_ext, my * HQ_LOCAL, HQ_LOCAL, axis=2)
    K_sh = jnp.transpose(K_sh, (0, 2, 1, 3)).astype(jnp.bfloat16)
    V_sh = jnp.transpose(V_sh, (0, 2, 1, 3)).astype(jnp.bfloat16)

    def body(x_ref, wq_ref, k_ref, v_ref, wo_ref, out_ref,
             ctx_ref, part_ref, fin_ref, scat_ref,
             send_a, recv_a, send_b, recv_b):
        me = lax.axis_index("i")

        barrier_sem = pltpu.get_barrier_semaphore()
        for d in range(1, N_DEV):
            pl.semaphore_signal(
                barrier_sem, inc=1,
                device_id=((me + d) % N_DEV,),
                device_id_type=pl.DeviceIdType.MESH,
            )

        qi = lax.broadcasted_iota(jnp.int32, (SQ, SKV), 0)
        ki = lax.broadcasted_iota(jnp.int32, (SQ, SKV), 1)
        bias = jnp.where(jnp.abs(qi - ki) <= WINDOW, 0.0, -60.0)

        wq = wq_ref[:, :].astype(jnp.bfloat16)
        wo = wo_ref[:, :].astype(jnp.bfloat16)

        x2d = x_ref[:].reshape(B * SQ, DMODEL).astype(jnp.bfloat16)
        q_all = jnp.dot(x2d, wq, preferred_element_type=jnp.float32)
        q16 = (q_all * SCALE).astype(jnp.bfloat16)

        for b in range(B):
            for h in range(HQ_LOCAL):
                q_bh = q16[b * SQ:(b + 1) * SQ, h * DH:(h + 1) * DH]
                s = lax.dot_general(
                    q_bh, k_ref[b, h], (((1,), (1,)), ((), ())),
                    preferred_element_type=jnp.float32,
                )
                w = jnp.exp(s + bias)
                rnorm = 1.0 / jnp.sum(w, axis=1, keepdims=True)
                ctx = jnp.dot(
                    w.astype(jnp.bfloat16), v_ref[b, h],
                    preferred_element_type=jnp.float32,
                )
                ctx_ref[b, :, h * DH:(h + 1) * DH] = (
                    ctx * rnorm
                ).astype(jnp.bfloat16)

        ctx2d = ctx_ref[:].reshape(B * SQ, HQ_LOCAL * DH)
        p_all = jnp.dot(ctx2d, wo, preferred_element_type=jnp.float32)
        out_ref[:] = p_all.reshape(B, SQ, DMODEL)
        part_ref[:] = p_all.astype(jnp.bfloat16).reshape(B, SQ, DMODEL)

        my_rows = pl.ds(me * QR, QR)

        pl.semaphore_wait(barrier_sem, N_DEV - 1)

        sends = []
        for c in range(CH):
            cols = pl.ds(c * CW, CW)
            for d in range(1, N_DEV):
                t = (me + d) % N_DEV
                rdma = pltpu.make_async_remote_copy(
                    src_ref=part_ref.at[:, pl.ds(t * QR, QR), cols],
                    dst_ref=scat_ref.at[N_DEV - 1 - d, :, :, cols],
                    send_sem=send_a.at[c * NP + d - 1],
                    recv_sem=recv_a.at[c * NP + N_DEV - 1 - d],
                    device_id=(t,),
                    device_id_type=pl.DeviceIdType.MESH,
                )
                rdma.start()
                sends.append(rdma)

        for c in range(CH):
            cols = pl.ds(c * CW, CW)
            for r in range(NP):
                pltpu.make_async_remote_copy(
                    src_ref=scat_ref.at[r, :, :, cols],
                    dst_ref=scat_ref.at[r, :, :, cols],
                    send_sem=send_a.at[c * NP + r],
                    recv_sem=recv_a.at[c * NP + r],
                    device_id=(me,), device_id_type=pl.DeviceIdType.MESH,
                ).wait_recv()
            red = (
                out_ref[:, my_rows, cols]
                + scat_ref[0, :, :, cols].astype(jnp.float32)
                + scat_ref[1, :, :, cols].astype(jnp.float32)
                + scat_ref[2, :, :, cols].astype(jnp.float32)
            )
            out_ref[:, my_rows, cols] = red
            fin_ref[:, my_rows, cols] = red.astype(jnp.bfloat16)
            for d in range(1, N_DEV):
                t = (me + d) % N_DEV
                rdma = pltpu.make_async_remote_copy(
                    src_ref=fin_ref.at[:, my_rows, cols],
                    dst_ref=fin_ref.at[:, my_rows, cols],
                    send_sem=send_b.at[c * NP + d - 1],
                    recv_sem=recv_b.at[c * NP + N_DEV - 1 - d],
                    device_id=(t,),
                    device_id_type=pl.DeviceIdType.MESH,
                )
                rdma.start()
                sends.append(rdma)

        for c in range(CH):
            cols = pl.ds(c * CW, CW)
            for r in range(NP):
                s = (me + 1 + r) % N_DEV
                pltpu.make_async_remote_copy(
                    src_ref=fin_ref.at[:, pl.ds(s * QR, QR), cols],
                    dst_ref=fin_ref.at[:, pl.ds(s * QR, QR), cols],
                    send_sem=send_b.at[c * NP + r],
                    recv_sem=recv_b.at[c * NP + r],
                    device_id=(me,), device_id_type=pl.DeviceIdType.MESH,
                ).wait_recv()
        for r in range(NP):
            s_rows = pl.ds(((me + 1 + r) % N_DEV) * QR, QR)
            out_ref[:, s_rows, :] = fin_ref[:, s_rows, :].astype(jnp.float32)
        for rdma in sends:
            rdma.wait_send()

    return pl.pallas_call(
        body,
        out_shape=jax.ShapeDtypeStruct((B, SQ, DMODEL), jnp.float32),
        in_specs=[pl.BlockSpec(memory_space=pltpu.VMEM)] * 5,
        out_specs=pl.BlockSpec(memory_space=pltpu.VMEM),
        scratch_shapes=[
            pltpu.VMEM((B, SQ, HQ_LOCAL * DH), jnp.bfloat16),
            pltpu.VMEM((B, SQ, DMODEL), jnp.bfloat16),
            pltpu.VMEM((B, SQ, DMODEL), jnp.bfloat16),
            pltpu.VMEM((NP, B, QR, DMODEL), jnp.bfloat16),
            pltpu.SemaphoreType.DMA((CH * NP,)),
            pltpu.SemaphoreType.DMA((CH * NP,)),
            pltpu.SemaphoreType.DMA((CH * NP,)),
            pltpu.SemaphoreType.DMA((CH * NP,)),
        ],
        compiler_params=pltpu.CompilerParams(collective_id=0),
    )(x, Wq, K_sh, V_sh, Wo)


# device time: 14869 ns/iter; 1.0595x vs baseline; 1.0226x over previous
import jax
import jax.numpy as jnp
from jax import lax
from jax.experimental import pallas as pl
from jax.experimental.pallas import tpu as pltpu

N_DEV = 4
B, SQ, SKV, HQ_LOCAL, DH = 2, 256, 256, 4, 64
DMODEL = 512
WINDOW = 128
SCALE = 0.125
CH = 2
QR = SQ // N_DEV
CW = DMODEL // CH
NP = N_DEV - 1


def kernel(x, Wq, K_ext, V_ext, Wo):
    my = lax.axis_index("i")
    K_sh = lax.dynamic_slice_in_dim(K_ext, my * HQ_LOCAL, HQ_LOCAL, axis=2)
    V_sh = lax.dynamic_slice_in_dim(V_ext, my * HQ_LOCAL, HQ_LOCAL, axis=2)
    K_sh = jnp.transpose(K_sh, (0, 2, 1, 3)).astype(jnp.bfloat16)
    V_sh = jnp.transpose(V_sh, (0, 2, 1, 3)).astype(jnp.bfloat16)

    def body(x_ref, wq_ref, k_ref, v_ref, wo_ref, out_ref,
             ctx_ref, part_ref, fin_ref, scat_ref,
             scale_src, scale_scat,
             send_a, recv_a, send_b, recv_b, send_s, recv_s):
        me = lax.axis_index("i")

        barrier_sem = pltpu.get_barrier_semaphore()
        for d in range(1, N_DEV):
            pl.semaphore_signal(
                barrier_sem, inc=1,
                device_id=((me + d) % N_DEV,),
                device_id_type=pl.DeviceIdType.MESH,
            )

        qi = lax.broadcasted_iota(jnp.int32, (SQ, SKV), 0)
        ki = lax.broadcasted_iota(jnp.int32, (SQ, SKV), 1)
        bias = jnp.where(jnp.abs(qi - ki) <= WINDOW, 0.0, -60.0)

        wq = wq_ref[:, :].astype(jnp.bfloat16)
        wo = wo_ref[:, :].astype(jnp.bfloat16)

        x2d = x_ref[:].reshape(B * SQ, DMODEL).astype(jnp.bfloat16)
        q_all = jnp.dot(x2d, wq, preferred_element_type=jnp.float32)
        q16 = (q_all * SCALE).astype(jnp.bfloat16)

        for b in range(B):
            for h in range(HQ_LOCAL):
                q_bh = q16[b * SQ:(b + 1) * SQ, h * DH:(h + 1) * DH]
                s = lax.dot_general(
                    q_bh, k_ref[b, h], (((1,), (1,)), ((), ())),
                    preferred_element_type=jnp.float32,
                )
                w = jnp.exp(s + bias)
                rnorm = 1.0 / jnp.sum(w, axis=1, keepdims=True)
                ctx = jnp.dot(
                    w.astype(jnp.bfloat16), v_ref[b, h],
                    preferred_element_type=jnp.float32,
                )
                ctx_ref[b, :, h * DH:(h + 1) * DH] = (
                    ctx * rnorm
                ).astype(jnp.bfloat16)

        ctx2d = ctx_ref[:].reshape(B * SQ, HQ_LOCAL * DH)
        p_all = jnp.dot(ctx2d, wo, preferred_element_type=jnp.float32)
        out_ref[:] = p_all.reshape(B, SQ, DMODEL)

        for d in range(1, N_DEV):
            rows = pl.ds(((me + d) % N_DEV) * QR, QR)
            blk = out_ref[:, rows, :]
            m = jnp.max(jnp.abs(blk)) + 1e-20
            scale_src[d - 1] = jnp.full((8, 128), m * (1.0 / 127.0), jnp.float32)
            part_ref[:, rows, :] = jnp.round(blk * (127.0 / m)).astype(jnp.int8)

        my_rows = pl.ds(me * QR, QR)

        pl.semaphore_wait(barrier_sem, N_DEV - 1)

        sends = []
        for d in range(1, N_DEV):
            t = (me + d) % N_DEV
            rs = pltpu.make_async_remote_copy(
                src_ref=scale_src.at[d - 1],
                dst_ref=scale_scat.at[N_DEV - 1 - d],
                send_sem=send_s.at[d - 1],
                recv_sem=recv_s.at[N_DEV - 1 - d],
                device_id=(t,),
                device_id_type=pl.DeviceIdType.MESH,
            )
            rs.start()
            sends.append(rs)
        for c in range(CH):
            cols = pl.ds(c * CW, CW)
            for d in range(1, N_DEV):
                t = (me + d) % N_DEV
                rdma = pltpu.make_async_remote_copy(
                    src_ref=part_ref.at[:, pl.ds(t * QR, QR), cols],
                    dst_ref=scat_ref.at[N_DEV - 1 - d, :, :, cols],
                    send_sem=send_a.at[c * NP + d - 1],
                    recv_sem=recv_a.at[c * NP + N_DEV - 1 - d],
                    device_id=(t,),
                    device_id_type=pl.DeviceIdType.MESH,
                )
                rdma.start()
                sends.append(rdma)

        for r in range(NP):
            pltpu.make_async_remote_copy(
                src_ref=scale_scat.at[r], dst_ref=scale_scat.at[r],
                send_sem=send_s.at[r], recv_sem=recv_s.at[r],
                device_id=(me,), device_id_type=pl.DeviceIdType.MESH,
            ).wait_recv()
        sc = [scale_scat[r, :1, :1] for r in range(NP)]
        for c in range(CH):
            cols = pl.ds(c * CW, CW)
            for r in range(NP):
                pltpu.make_async_remote_copy(
                    src_ref=scat_ref.at[r, :, :, cols],
                    dst_ref=scat_ref.at[r, :, :, cols],
                    send_sem=send_a.at[c * NP + r],
                    recv_sem=recv_a.at[c * NP + r],
                    device_id=(me,), device_id_type=pl.DeviceIdType.MESH,
                ).wait_recv()
            red = (
                out_ref[:, my_rows, cols]
                + scat_ref[0, :, :, cols].astype(jnp.float32) * sc[0]
                + scat_ref[1, :, :, cols].astype(jnp.float32) * sc[1]
                + scat_ref[2, :, :, cols].astype(jnp.float32) * sc[2]
            )
            out_ref[:, my_rows, cols] = red
            fin_ref[:, my_rows, cols] = red.astype(jnp.bfloat16)
            for d in range(1, N_DEV):
                t = (me + d) % N_DEV
                rdma = pltpu.make_async_remote_copy(
                    src_ref=fin_ref.at[:, my_rows, cols],
                    dst_ref=fin_ref.at[:, my_rows, cols],
                    send_sem=send_b.at[c * NP + d - 1],
                    recv_sem=recv_b.at[c * NP + N_DEV - 1 - d],
                    device_id=(t,),
                    device_id_type=pl.DeviceIdType.MESH,
                )
                rdma.start()
                sends.append(rdma)

        for c in range(CH):
            cols = pl.ds(c * CW, CW)
            for r in range(NP):
                s = (me + 1 + r) % N_DEV
                pltpu.make_async_remote_copy(
                    src_ref=fin_ref.at[:, pl.ds(s * QR, QR), cols],
                    dst_ref=fin_ref.at[:, pl.ds(s * QR, QR), cols],
                    send_sem=send_b.at[c * NP + r],
                    recv_sem=recv_b.at[c * NP + r],
                    device_id=(me,), device_id_type=pl.DeviceIdType.MESH,
                ).wait_recv()
        for r in range(NP):
            s_rows = pl.ds(((me + 1 + r) % N_DEV) * QR, QR)
            out_ref[:, s_rows, :] = fin_ref[:, s_rows, :].astype(jnp.float32)
        for rdma in sends:
            rdma.wait_send()

    return pl.pallas_call(
        body,
        out_shape=jax.ShapeDtypeStruct((B, SQ, DMODEL), jnp.float32),
        in_specs=[pl.BlockSpec(memory_space=pltpu.VMEM)] * 5,
        out_specs=pl.BlockSpec(memory_space=pltpu.VMEM),
        scratch_shapes=[
            pltpu.VMEM((B, SQ, HQ_LOCAL * DH), jnp.bfloat16),
            pltpu.VMEM((B, SQ, DMODEL), jnp.int8),
            pltpu.VMEM((B, SQ, DMODEL), jnp.bfloat16),
            pltpu.VMEM((NP, B, QR, DMODEL), jnp.int8),
            pltpu.VMEM((NP, 8, 128), jnp.float32),
            pltpu.VMEM((NP, 8, 128), jnp.float32),
            pltpu.SemaphoreType.DMA((CH * NP,)),
            pltpu.SemaphoreType.DMA((CH * NP,)),
            pltpu.SemaphoreType.DMA((CH * NP,)),
            pltpu.SemaphoreType.DMA((CH * NP,)),
            pltpu.SemaphoreType.DMA((NP,)),
            pltpu.SemaphoreType.DMA((NP,)),
        ],
        compiler_params=pltpu.CompilerParams(collective_id=0),
    )(x, Wq, K_sh, V_sh, Wo)
